# baseline (device time: 1234287 ns/iter reference)
import jax
import jax.numpy as jnp
from jax import lax
from jax.experimental import pallas as pl
from jax.experimental.pallas import tpu as pltpu

N_DEV = 4


def _fused_matmul_reduce_scatter(A, Wo, B, S, SB, N):
    M, K = A.shape
    n_hops = N_DEV - 1
    BN = 2048
    NN = N // BN
    NH = N // 2
    NNH = NH // BN
    RB = B * SB

    def body(a_ref, w_ref, out_ref, recv_ref, part_ref,
             send_sems_a, recv_sems_a, send_sems_b, recv_sems_b):
        my = lax.axis_index("i")
        left = lax.rem(my + N_DEV - 1, N_DEV)
        right = lax.rem(my + 1, N_DEV)

        def rows(bb):
            return pl.ds(bb * SB, SB)

        def mm(a_vr, w_vr, o_vr):
            o_vr[...] = jnp.dot(
                a_vr[...], w_vr[...], preferred_element_type=jnp.float32
            )

        def compute_block(b, dst_ref):
            pltpu.emit_pipeline(
                mm,
                grid=(NN, B),
                in_specs=[
                    pl.BlockSpec((SB, K), lambda nn, bb: (bb * N_DEV + b, 0)),
                    pl.BlockSpec((K, BN), lambda nn, bb: (0, nn)),
                ],
                out_specs=[pl.BlockSpec((SB, BN), lambda nn, bb: (bb, nn))],
            )(a_ref, w_ref, dst_ref)

        def compute_half_block(b, off, dst_ref):
            pltpu.emit_pipeline(
                mm,
                grid=(NNH, B),
                in_specs=[
                    pl.BlockSpec((SB, K), lambda nn, bb: (bb * N_DEV + b, 0)),
                    pl.BlockSpec((K, BN), lambda nn, bb: (0, nn)),
                ],
                out_specs=[pl.BlockSpec((SB, BN), lambda nn, bb: (bb, nn))],
            )(a_ref, w_ref.at[:, pl.ds(off, NH)],
              dst_ref.at[:, pl.ds(off, NH)])

        def compute_chunk_half(b, bb, off, dst_ref):
            pltpu.emit_pipeline(
                mm,
                grid=(NNH,),
                in_specs=[
                    pl.BlockSpec((SB, K), lambda nn: (bb * N_DEV + b, 0)),
                    pl.BlockSpec((K, BN), lambda nn: (0, nn)),
                ],
                out_specs=[pl.BlockSpec((SB, BN), lambda nn: (0, nn))],
            )(a_ref, w_ref.at[:, pl.ds(off, NH)],
              dst_ref.at[rows(bb), pl.ds(off, NH)])

        def compute_chunk_full(b, bb, dst_ref):
            pltpu.emit_pipeline(
                mm,
                grid=(NN,),
                in_specs=[
                    pl.BlockSpec((SB, K), lambda nn: (bb * N_DEV + b, 0)),
                    pl.BlockSpec((K, BN), lambda nn: (0, nn)),
                ],
                out_specs=[pl.BlockSpec((SB, BN), lambda nn: (0, nn))],
            )(a_ref, w_ref, dst_ref.at[rows(bb), :])

        def accumulate_chunk(src_chunk_ref, part_chunk_ref, dst_chunk_ref):
            def add(r_vr, p_vr, o_vr):
                o_vr[...] = r_vr[...] + p_vr[...]

            pltpu.emit_pipeline(
                add,
                grid=(NN,),
                in_specs=[
                    pl.BlockSpec((SB, BN), lambda nn: (0, nn)),
                    pl.BlockSpec((SB, BN), lambda nn: (0, nn)),
                ],
                out_specs=[pl.BlockSpec((SB, BN), lambda nn: (0, nn))],
            )(src_chunk_ref, part_chunk_ref, dst_chunk_ref)

        def send_chunk_a(t, bb, src_ref):
            rdma = pltpu.make_async_remote_copy(
                src_ref=src_ref.at[rows(bb), pl.ds(0, NH)],
                dst_ref=recv_ref.at[t, rows(bb), pl.ds(0, NH)],
                send_sem=send_sems_a.at[t, bb],
                recv_sem=recv_sems_a.at[t, bb],
                device_id=(right,),
                device_id_type=pl.DeviceIdType.MESH,
            )
            rdma.start()
            return rdma

        def send_chunk_b(t, bb, src_ref):
            rdma = pltpu.make_async_remote_copy(
                src_ref=src_ref.at[rows(bb), pl.ds(NH, NH)],
                dst_ref=recv_ref.at[t, rows(bb), pl.ds(NH, NH)],
                send_sem=send_sems_b.at[t, bb],
                recv_sem=recv_sems_b.at[t, bb],
                device_id=(left,),
                device_id_type=pl.DeviceIdType.MESH,
            )
            rdma.start()
            return rdma

        b_cw0 = lax.rem(my + 3, N_DEV)
        b_ccw0 = lax.rem(my + 1, N_DEV)
        b_h0 = lax.rem(my + 2, N_DEV)
        b_h2 = my

        barrier = pltpu.get_barrier_semaphore()
        for nbr in (left, right):
            pl.semaphore_signal(
                barrier, inc=1,
                device_id=(nbr,), device_id_type=pl.DeviceIdType.MESH,
            )
        pl.semaphore_wait(barrier, 2)

        rdmas_a, rdmas_b = [], []
        for bb in range(B):
            compute_chunk_half(b_cw0, bb, 0, part_ref.at[0])
            compute_chunk_half(b_ccw0, bb, NH, part_ref.at[0])
            rdmas_a.append(send_chunk_a(0, bb, part_ref.at[0]))
            rdmas_b.append(send_chunk_b(0, bb, part_ref.at[0]))
            compute_chunk_full(b_h0, bb, part_ref.at[1])

        for t in range(n_hops):
            last = t == n_hops - 1
            next_a, next_b = [], []
            for bb in range(B):
                rdmas_a[bb].wait()
                rdmas_b[bb].wait()
                dst = out_ref if last else recv_ref.at[t]
                accumulate_chunk(
                    recv_ref.at[t, rows(bb), :],
                    part_ref.at[(t + 1) % 2, rows(bb), :],
                    dst.at[rows(bb), :],
                )
                if not last:
                    next_a.append(send_chunk_a(t + 1, bb, recv_ref.at[t]))
                    next_b.append(send_chunk_b(t + 1, bb, recv_ref.at[t]))
                if t == 0:
                    compute_chunk_half(b_ccw0, bb, 0, part_ref.at[0])
                    compute_chunk_half(b_cw0, bb, NH, part_ref.at[0])
                elif t == 1:
                    compute_chunk_full(b_h2, bb, part_ref.at[1])
            rdmas_a, rdmas_b = next_a, next_b

    out, _, _ = pl.pallas_call(
        body,
        out_shape=[
            jax.ShapeDtypeStruct((RB, N), jnp.float32),
            jax.ShapeDtypeStruct((n_hops, RB, N), jnp.float32),
            jax.ShapeDtypeStruct((2, RB, N), jnp.float32),
        ],
        in_specs=[
            pl.BlockSpec(memory_space=pl.ANY),
            pl.BlockSpec(memory_space=pl.ANY),
        ],
        out_specs=[
            pl.BlockSpec(memory_space=pl.ANY),
            pl.BlockSpec(memory_space=pl.ANY),
            pl.BlockSpec(memory_space=pl.ANY),
        ],
        scratch_shapes=[
            pltpu.SemaphoreType.DMA((n_hops, B)),
            pltpu.SemaphoreType.DMA((n_hops, B)),
            pltpu.SemaphoreType.DMA((n_hops, B)),
            pltpu.SemaphoreType.DMA((n_hops, B)),
        ],
        compiler_params=pltpu.CompilerParams(
            collective_id=0,
            vmem_limit_bytes=100 * 1024 * 1024,
        ),
    )(A, Wo)
    return out


def kernel(O, Wo):
    B, S, HL, D = O.shape
    K, N = Wo.shape
    assert HL * D == K
    A = O.reshape(B * S, K)
    SB = S // N_DEV
    out = _fused_matmul_reduce_scatter(A, Wo, B=B, S=S, SB=SB, N=N)
    return out.reshape(B, SB, N)
